# baseline (speedup 1.0000x reference)
import functools

import jax
import jax.numpy as jnp
from jax import lax
from jax.experimental import pallas as pl
from jax.experimental.pallas import tpu as pltpu
from jax.experimental.pallas import tpu_sc as plsc

NW = 32
NS = 16
CHUNK = 128
GW = 64


def _sc_mesh():
    return plsc.VectorSubcoreMesh(core_axis_name="c", subcore_axis_name="s")


def _make_deg_kernel(Npad, E):
    EW = E // NW

    @functools.partial(
        pl.kernel,
        out_type=jax.ShapeDtypeStruct((NW, Npad), jnp.float32),
        mesh=_sc_mesh(),
        scratch_types=[
            pltpu.VMEM((EW,), jnp.int32),
            pltpu.VMEM((Npad,), jnp.float32),
        ],
        compiler_params=pltpu.CompilerParams(needs_layout_passes=False),
    )
    def deg_kernel(ei_hbm, out_hbm, dst_v, hist_v):
        cid = lax.axis_index("c")
        sid = lax.axis_index("s")
        wid = sid * 2 + cid
        pltpu.sync_copy(ei_hbm.at[pl.ds(E + wid * EW, EW)], dst_v)

        def zbody(i, carry):
            hist_v[pl.ds(i * 16, 16)] = jnp.zeros((16,), jnp.float32)
            return carry

        lax.fori_loop(0, Npad // 16, zbody, 0)

        ones = jnp.ones((16,), jnp.float32)

        def hbody(i, carry):
            idx = dst_v[pl.ds(i * 16, 16)]
            plsc.addupdate_scatter(hist_v, [idx], ones)
            return carry

        lax.fori_loop(0, EW // 16, hbody, 0)
        pltpu.sync_copy(hist_v, out_hbm.at[wid])

    return deg_kernel


def _make_agg_kernel(Npad, Hdim, K):
    stripe = Npad // NS

    @functools.partial(
        pl.kernel,
        out_type=jax.ShapeDtypeStruct((2, Npad, Hdim), jnp.float32),
        mesh=_sc_mesh(),
        scratch_types=[
            pltpu.VMEM((K * CHUNK,), jnp.int32),
            pltpu.VMEM((3, 2, CHUNK), jnp.int32),
            pltpu.VMEM((CHUNK, Hdim), jnp.float32),
            pltpu.VMEM((CHUNK, Hdim), jnp.float32),
            pltpu.VMEM((CHUNK, Hdim), jnp.float32),
            pltpu.VMEM_SHARED((Npad, Hdim), jnp.float32),
            pltpu.VMEM_SHARED((Npad, Hdim), jnp.float32),
            pltpu.SemaphoreType.DMA,
            pltpu.SemaphoreType.DMA,
            pltpu.SemaphoreType.DMA,
        ],
        compiler_params=pltpu.CompilerParams(use_tc_tiling_on_sc=False),
    )
    def agg_kernel(u_hbm, pk_hbm, zeros_hbm, out_hbm,
                   pk_v, ib, rows0, rows1, rows2, u_s, acc,
                   sem0, sem1, sem2):
        cid = lax.axis_index("c")
        sid = lax.axis_index("s")
        wid = sid * 2 + cid
        pltpu.sync_copy(pk_hbm.at[pl.ds(wid * K * CHUNK, K * CHUNK)], pk_v)
        pltpu.sync_copy(u_hbm.at[pl.ds(sid * stripe, stripe)],
                        u_s.at[pl.ds(sid * stripe, stripe)])
        pltpu.sync_copy(zeros_hbm.at[pl.ds(sid * stripe, stripe)],
                        acc.at[pl.ds(sid * stripe, stripe)])

        def unpack(j, p):
            for v in range(CHUNK // 16):
                pk = pk_v[pl.ds(j * CHUNK + v * 16, 16)]
                ib[p, 0, pl.ds(v * 16, 16)] = pk & 0xFFFF
                ib[p, 1, pl.ds(v * 16, 16)] = lax.shift_right_logical(pk, 16)

        unpack(0, 0)
        unpack(1, 1)
        plsc.subcore_barrier()
        pltpu.async_copy(u_s.at[ib.at[0, 0]], rows0, sem0)
        pltpu.async_copy(u_s.at[ib.at[1, 0]], rows1, sem1)

        def body(j, carry):
            def phase(rc, sc_, rn, sn, p, pn, src_now, src_next):
                pltpu.make_async_copy(src_now.at[ib.at[p, 0]], rc, sc_).wait()

                @pl.when(j + 2 < K)
                def _pre():
                    unpack(j + 2, pn)
                    pltpu.async_copy(src_next.at[ib.at[pn, 0]], rn, sn)

                pltpu.sync_copy(rc, acc.at[ib.at[p, 1]], add=True)

            @pl.when(j % 3 == 0)
            def _p0():
                phase(rows0, sem0, rows2, sem2, 0, 2, u_s, u_hbm)

            @pl.when(j % 3 == 1)
            def _p1():
                phase(rows1, sem1, rows0, sem0, 1, 0, u_s, u_s)

            @pl.when(j % 3 == 2)
            def _p2():
                phase(rows2, sem2, rows1, sem1, 2, 1, u_hbm, u_s)

            return carry

        lax.fori_loop(0, K, body, 0)
        plsc.subcore_barrier()
        pltpu.sync_copy(acc.at[pl.ds(sid * stripe, stripe)],
                        out_hbm.at[cid, pl.ds(sid * stripe, stripe)])

    return agg_kernel


def _make_agg2g_kernel(Npad, Hdim, K2):
    stripe = Npad // NS
    KA = (K2 + 1) // 2
    KB = K2 - KA

    @functools.partial(
        pl.kernel,
        out_type=[jax.ShapeDtypeStruct((Npad, Hdim), jnp.float32)] * 2,
        mesh=_sc_mesh(),
        scratch_types=[
            pltpu.VMEM((KA * CHUNK,), jnp.int32),
            pltpu.VMEM((3, 2, CHUNK), jnp.int32),
            pltpu.VMEM((CHUNK, Hdim), jnp.float32),
            pltpu.VMEM((CHUNK, Hdim), jnp.float32),
            pltpu.VMEM((CHUNK, Hdim), jnp.float32),
            pltpu.VMEM_SHARED((Npad, Hdim), jnp.float32),
            pltpu.VMEM_SHARED((Npad, Hdim), jnp.float32),
            pltpu.SemaphoreType.DMA,
            pltpu.SemaphoreType.DMA,
            pltpu.SemaphoreType.DMA,
        ],
        compiler_params=pltpu.CompilerParams(use_tc_tiling_on_sc=False),
    )
    def agg2g_kernel(ua_hbm, ub_hbm, pk_hbm, zeros_hbm, pa_hbm, pb_hbm,
                     pk_v, ib, rows0, rows1, rows2, u_s, acc,
                     sem0, sem1, sem2):
        cid = lax.axis_index("c")
        sid = lax.axis_index("s")
        pltpu.sync_copy(zeros_hbm.at[pl.ds(sid * stripe, stripe)],
                        acc.at[pl.ds(sid * stripe, stripe)])

        def unpack(j, p):
            for v in range(CHUNK // 16):
                pk = pk_v[pl.ds(j * CHUNK + v * 16, 16)]
                ib[p, 0, pl.ds(v * 16, 16)] = pk & 0xFFFF
                ib[p, 1, pl.ds(v * 16, 16)] = lax.shift_right_logical(pk, 16)

        def run(u_hbm, out_hbm):
            pltpu.sync_copy(u_hbm.at[pl.ds(sid * stripe, stripe)],
                            u_s.at[pl.ds(sid * stripe, stripe)])
            plsc.subcore_barrier()
            for j0, kn in ((0, KA), (KA, KB)):
                pltpu.sync_copy(
                    pk_hbm.at[pl.ds((sid * K2 + j0) * CHUNK, kn * CHUNK)],
                    pk_v.at[pl.ds(0, kn * CHUNK)])
                unpack(0, 0)
                unpack(1, 1)
                pltpu.async_copy(u_s.at[ib.at[0, 0]], rows0, sem0)
                pltpu.async_copy(u_s.at[ib.at[1, 0]], rows1, sem1)

                def body(j, carry):
                    def phase(rc, sc_, rn, sn, p, pn, src_now, src_next):
                        pltpu.make_async_copy(
                            src_now.at[ib.at[p, 0]], rc, sc_).wait()

                        @pl.when(j + 2 < kn)
                        def _pre():
                            unpack(j + 2, pn)
                            pltpu.async_copy(
                                src_next.at[ib.at[pn, 0]], rn, sn)

                        pltpu.sync_copy(rc, acc.at[ib.at[p, 1]], add=True)

                    @pl.when(j % 3 == 0)
                    def _p0():
                        phase(rows0, sem0, rows2, sem2, 0, 2, u_s, u_hbm)

                    @pl.when(j % 3 == 1)
                    def _p1():
                        phase(rows1, sem1, rows0, sem0, 1, 0, u_s, u_s)

                    @pl.when(j % 3 == 2)
                    def _p2():
                        phase(rows2, sem2, rows1, sem1, 2, 1, u_hbm, u_s)

                    return carry

                lax.fori_loop(0, kn, body, 0)
            plsc.subcore_barrier()
            pltpu.sync_copy(acc.at[pl.ds(sid * stripe, stripe)],
                            out_hbm.at[pl.ds(sid * stripe, stripe)])

        @pl.when(cid == 0)
        def _a():
            run(ua_hbm, pa_hbm)

        @pl.when(cid == 1)
        def _b():
            run(ub_hbm, pb_hbm)

    return agg2g_kernel


def _prep_body(dp_ref, o_ref, *, N):
    d = jnp.sum(dp_ref[...], axis=0) + 1.0
    nr, nc = d.shape
    ids = (lax.broadcasted_iota(jnp.int32, (nr, nc), 0) * nc
           + lax.broadcasted_iota(jnp.int32, (nr, nc), 1))
    o_ref[...] = jnp.where(ids < N, lax.rsqrt(d), 0.0)


def _mm1_body(x_ref, w_ref, dv_ref, *o_refs):
    z = jnp.dot(x_ref[...], w_ref[...], preferred_element_type=jnp.float32)
    u = dv_ref[...] * z
    for g, o_ref in enumerate(o_refs):
        o_ref[...] = u[:, g * GW:(g + 1) * GW]


def _mm2_body(*refs, G, O, full_parts):
    dv_ref, b_ref, w_ref, o_ref = refs[-4:]
    parts = []
    for g in range(G):
        p_ref, u_ref = refs[2 * g], refs[2 * g + 1]
        if full_parts:
            parts.append(p_ref[...] + u_ref[...])
        else:
            parts.append(p_ref[0] + p_ref[1] + u_ref[...])
    s = jnp.concatenate(parts, axis=1) if G > 1 else parts[0]
    h = jnp.maximum(dv_ref[...] * s + b_ref[...], 0.0)
    z = jnp.dot(h, w_ref[...], preferred_element_type=jnp.float32)
    u = dv_ref[...] * z
    o_ref[...] = u


def _fin_body(q_ref, u_ref, dv_ref, b_ref, o_ref):
    s = q_ref[0] + q_ref[1] + u_ref[...]
    o_ref[...] = dv_ref[...] * s + b_ref[...]


def kernel(x, edge_index, W1, b1, W2, b2):
    N, D = x.shape
    E = edge_index.shape[1]
    H = W1.shape[1]
    O = W2.shape[1]
    G1 = H // GW

    Npad = ((N + 1 + 255) // 256) * 256
    RB = Npad // 8
    K = -(-E // (NW * CHUNK))
    EP = NW * K * CHUNK
    EW = K * CHUNK

    pk_flat = jnp.concatenate([
        edge_index[0].astype(jnp.int32)
        | (edge_index[1].astype(jnp.int32) << 16),
        jnp.full((EP - E,), N << 16, jnp.int32),
    ])

    x_pad = jnp.zeros((Npad, D), jnp.float32).at[:N].set(x)
    zeros_g = jnp.zeros((Npad, GW), jnp.float32)

    deg_part = _make_deg_kernel(Npad, E)(
        edge_index.astype(jnp.int32).reshape(2 * E))

    NR = Npad // 128
    dinv2d = pl.pallas_call(
        functools.partial(_prep_body, N=N),
        out_shape=jax.ShapeDtypeStruct((NR, 128), jnp.float32),
    )(deg_part.reshape(NW, NR, 128))
    dinv = dinv2d.reshape(Npad, 1)

    u1g = pl.pallas_call(
        _mm1_body,
        grid=(Npad // RB,),
        in_specs=[
            pl.BlockSpec((RB, D), lambda i: (i, 0)),
            pl.BlockSpec((D, H), lambda i: (0, 0)),
            pl.BlockSpec((RB, 1), lambda i: (i, 0)),
        ],
        out_specs=[pl.BlockSpec((RB, GW), lambda i: (i, 0))] * G1,
        out_shape=[jax.ShapeDtypeStruct((Npad, GW), jnp.float32)] * G1,
    )(x_pad, W1, dinv)

    full_parts = G1 == 2
    if full_parts:
        K2 = 2 * K
        pg = list(_make_agg2g_kernel(Npad, GW, K2)(
            u1g[0], u1g[1], pk_flat, zeros_g))
    else:
        agg1 = _make_agg_kernel(Npad, GW, K)
        pg = [agg1(u1g[g], pk_flat, zeros_g) for g in range(G1)]

    mm2_ins = []
    mm2_specs = []
    for g in range(G1):
        mm2_ins += [pg[g], u1g[g]]
        if full_parts:
            mm2_specs += [pl.BlockSpec((RB, GW), lambda i: (i, 0))]
        else:
            mm2_specs += [pl.BlockSpec((2, RB, GW), lambda i: (0, i, 0))]
        mm2_specs += [pl.BlockSpec((RB, GW), lambda i: (i, 0))]
    mm2_ins += [dinv, b1.reshape(1, H), W2]
    mm2_specs += [pl.BlockSpec((RB, 1), lambda i: (i, 0)),
                  pl.BlockSpec((1, H), lambda i: (0, 0)),
                  pl.BlockSpec((H, O), lambda i: (0, 0))]
    u2 = pl.pallas_call(
        functools.partial(_mm2_body, G=G1, O=O, full_parts=full_parts),
        grid=(Npad // RB,),
        in_specs=mm2_specs,
        out_specs=pl.BlockSpec((RB, O), lambda i: (i, 0)),
        out_shape=jax.ShapeDtypeStruct((Npad, O), jnp.float32),
    )(*mm2_ins)

    q = _make_agg_kernel(Npad, O, K)(u2, pk_flat, zeros_g)

    RF = N // 10
    out = pl.pallas_call(
        _fin_body,
        grid=(10,),
        in_specs=[
            pl.BlockSpec((2, RF, O), lambda i: (0, i, 0)),
            pl.BlockSpec((RF, O), lambda i: (i, 0)),
            pl.BlockSpec((RF, 1), lambda i: (i, 0)),
            pl.BlockSpec((1, O), lambda i: (0, 0)),
        ],
        out_specs=pl.BlockSpec((RF, O), lambda i: (i, 0)),
        out_shape=jax.ShapeDtypeStruct((N, O), jnp.float32),
    )(q, u2, dinv, b2.reshape(1, O))

    return out


# speedup vs baseline: 1.0774x; 1.0774x over previous
import functools

import jax
import jax.numpy as jnp
from jax import lax
from jax.experimental import pallas as pl
from jax.experimental.pallas import tpu as pltpu
from jax.experimental.pallas import tpu_sc as plsc

NW = 32
NS = 16
CHUNK = 128
GW = 64


def _sc_mesh():
    return plsc.VectorSubcoreMesh(core_axis_name="c", subcore_axis_name="s")


def _make_deg_kernel(Npad, E):
    EW = E // NW

    @functools.partial(
        pl.kernel,
        out_type=jax.ShapeDtypeStruct((NW, Npad), jnp.float32),
        mesh=_sc_mesh(),
        scratch_types=[
            pltpu.VMEM((EW,), jnp.int32),
            pltpu.VMEM((Npad,), jnp.float32),
        ],
        compiler_params=pltpu.CompilerParams(needs_layout_passes=False),
    )
    def deg_kernel(ei_hbm, out_hbm, dst_v, hist_v):
        cid = lax.axis_index("c")
        sid = lax.axis_index("s")
        wid = sid * 2 + cid
        pltpu.sync_copy(ei_hbm.at[pl.ds(E + wid * EW, EW)], dst_v)

        def zbody(i, carry):
            hist_v[pl.ds(i * 16, 16)] = jnp.zeros((16,), jnp.float32)
            return carry

        lax.fori_loop(0, Npad // 16, zbody, 0)

        ones = jnp.ones((16,), jnp.float32)

        def hbody(i, carry):
            idx = dst_v[pl.ds(i * 16, 16)]
            plsc.addupdate_scatter(hist_v, [idx], ones)
            return carry

        lax.fori_loop(0, EW // 16, hbody, 0)
        pltpu.sync_copy(hist_v, out_hbm.at[wid])

    return deg_kernel


def _make_agg_kernel(Npad, Hdim, K):
    stripe = Npad // NS

    @functools.partial(
        pl.kernel,
        out_type=jax.ShapeDtypeStruct((2, Npad, Hdim), jnp.float32),
        mesh=_sc_mesh(),
        scratch_types=[
            pltpu.VMEM((K * CHUNK,), jnp.int32),
            pltpu.VMEM((3, 2, CHUNK), jnp.int32),
            pltpu.VMEM((CHUNK, Hdim), jnp.float32),
            pltpu.VMEM((CHUNK, Hdim), jnp.float32),
            pltpu.VMEM((CHUNK, Hdim), jnp.float32),
            pltpu.VMEM_SHARED((Npad, Hdim), jnp.float32),
            pltpu.VMEM_SHARED((Npad, Hdim), jnp.float32),
            pltpu.SemaphoreType.DMA,
            pltpu.SemaphoreType.DMA,
            pltpu.SemaphoreType.DMA,
        ],
        compiler_params=pltpu.CompilerParams(use_tc_tiling_on_sc=False),
    )
    def agg_kernel(u_hbm, pk_hbm, zeros_hbm, out_hbm,
                   pk_v, ib, rows0, rows1, rows2, u_s, acc,
                   sem0, sem1, sem2):
        cid = lax.axis_index("c")
        sid = lax.axis_index("s")
        wid = sid * 2 + cid
        pltpu.sync_copy(pk_hbm.at[pl.ds(wid * K * CHUNK, K * CHUNK)], pk_v)
        pltpu.sync_copy(u_hbm.at[pl.ds(sid * stripe, stripe)],
                        u_s.at[pl.ds(sid * stripe, stripe)])
        pltpu.sync_copy(zeros_hbm.at[pl.ds(sid * stripe, stripe)],
                        acc.at[pl.ds(sid * stripe, stripe)])

        def unpack(j, p):
            for v in range(CHUNK // 16):
                pk = pk_v[pl.ds(j * CHUNK + v * 16, 16)]
                ib[p, 0, pl.ds(v * 16, 16)] = pk & 0xFFFF
                ib[p, 1, pl.ds(v * 16, 16)] = lax.shift_right_logical(pk, 16)

        unpack(0, 0)
        unpack(1, 1)
        plsc.subcore_barrier()
        pltpu.async_copy(u_s.at[ib.at[0, 0]], rows0, sem0)
        pltpu.async_copy(u_s.at[ib.at[1, 0]], rows1, sem1)

        def body(j, carry):
            def phase(rc, sc_, rn, sn, p, pn, src_now, src_next):
                pltpu.make_async_copy(src_now.at[ib.at[p, 0]], rc, sc_).wait()

                @pl.when(j + 2 < K)
                def _pre():
                    unpack(j + 2, pn)
                    pltpu.async_copy(src_next.at[ib.at[pn, 0]], rn, sn)

                pltpu.sync_copy(rc, acc.at[ib.at[p, 1]], add=True)

            @pl.when(j % 3 == 0)
            def _p0():
                phase(rows0, sem0, rows2, sem2, 0, 2, u_s, u_s)

            @pl.when(j % 3 == 1)
            def _p1():
                phase(rows1, sem1, rows0, sem0, 1, 0, u_s, u_s)

            @pl.when(j % 3 == 2)
            def _p2():
                phase(rows2, sem2, rows1, sem1, 2, 1, u_s, u_s)

            return carry

        lax.fori_loop(0, K, body, 0)
        plsc.subcore_barrier()
        pltpu.sync_copy(acc.at[pl.ds(sid * stripe, stripe)],
                        out_hbm.at[cid, pl.ds(sid * stripe, stripe)])

    return agg_kernel


def _make_agg2g_kernel(Npad, Hdim, K2):
    stripe = Npad // NS
    KA = (K2 + 1) // 2
    KB = K2 - KA

    @functools.partial(
        pl.kernel,
        out_type=[jax.ShapeDtypeStruct((Npad, Hdim), jnp.float32)] * 2,
        mesh=_sc_mesh(),
        scratch_types=[
            pltpu.VMEM((KA * CHUNK,), jnp.int32),
            pltpu.VMEM((3, 2, CHUNK), jnp.int32),
            pltpu.VMEM((CHUNK, Hdim), jnp.float32),
            pltpu.VMEM((CHUNK, Hdim), jnp.float32),
            pltpu.VMEM((CHUNK, Hdim), jnp.float32),
            pltpu.VMEM_SHARED((Npad, Hdim), jnp.float32),
            pltpu.VMEM_SHARED((Npad, Hdim), jnp.float32),
            pltpu.SemaphoreType.DMA,
            pltpu.SemaphoreType.DMA,
            pltpu.SemaphoreType.DMA,
        ],
        compiler_params=pltpu.CompilerParams(use_tc_tiling_on_sc=False),
    )
    def agg2g_kernel(ua_hbm, ub_hbm, pk_hbm, zeros_hbm, pa_hbm, pb_hbm,
                     pk_v, ib, rows0, rows1, rows2, u_s, acc,
                     sem0, sem1, sem2):
        cid = lax.axis_index("c")
        sid = lax.axis_index("s")
        pltpu.sync_copy(zeros_hbm.at[pl.ds(sid * stripe, stripe)],
                        acc.at[pl.ds(sid * stripe, stripe)])

        def unpack(j, p):
            for v in range(CHUNK // 16):
                pk = pk_v[pl.ds(j * CHUNK + v * 16, 16)]
                ib[p, 0, pl.ds(v * 16, 16)] = pk & 0xFFFF
                ib[p, 1, pl.ds(v * 16, 16)] = lax.shift_right_logical(pk, 16)

        def run(u_hbm, out_hbm):
            pltpu.sync_copy(u_hbm.at[pl.ds(sid * stripe, stripe)],
                            u_s.at[pl.ds(sid * stripe, stripe)])
            plsc.subcore_barrier()
            for j0, kn in ((0, KA), (KA, KB)):
                pltpu.sync_copy(
                    pk_hbm.at[pl.ds((sid * K2 + j0) * CHUNK, kn * CHUNK)],
                    pk_v.at[pl.ds(0, kn * CHUNK)])
                unpack(0, 0)
                unpack(1, 1)
                pltpu.async_copy(u_s.at[ib.at[0, 0]], rows0, sem0)
                pltpu.async_copy(u_s.at[ib.at[1, 0]], rows1, sem1)

                def body(j, carry):
                    def phase(rc, sc_, rn, sn, p, pn, src_now, src_next):
                        pltpu.make_async_copy(
                            src_now.at[ib.at[p, 0]], rc, sc_).wait()

                        @pl.when(j + 2 < kn)
                        def _pre():
                            unpack(j + 2, pn)
                            pltpu.async_copy(
                                src_next.at[ib.at[pn, 0]], rn, sn)

                        pltpu.sync_copy(rc, acc.at[ib.at[p, 1]], add=True)

                    @pl.when(j % 3 == 0)
                    def _p0():
                        phase(rows0, sem0, rows2, sem2, 0, 2, u_s, u_s)

                    @pl.when(j % 3 == 1)
                    def _p1():
                        phase(rows1, sem1, rows0, sem0, 1, 0, u_s, u_s)

                    @pl.when(j % 3 == 2)
                    def _p2():
                        phase(rows2, sem2, rows1, sem1, 2, 1, u_s, u_s)

                    return carry

                lax.fori_loop(0, kn, body, 0)
            plsc.subcore_barrier()
            pltpu.sync_copy(acc.at[pl.ds(sid * stripe, stripe)],
                            out_hbm.at[pl.ds(sid * stripe, stripe)])

        @pl.when(cid == 0)
        def _a():
            run(ua_hbm, pa_hbm)

        @pl.when(cid == 1)
        def _b():
            run(ub_hbm, pb_hbm)

    return agg2g_kernel


def _prep_body(dp_ref, o_ref, *, N):
    d = jnp.sum(dp_ref[...], axis=0) + 1.0
    nr, nc = d.shape
    ids = (lax.broadcasted_iota(jnp.int32, (nr, nc), 0) * nc
           + lax.broadcasted_iota(jnp.int32, (nr, nc), 1))
    o_ref[...] = jnp.where(ids < N, lax.rsqrt(d), 0.0)


def _mm1_body(x_ref, w_ref, dv_ref, *o_refs):
    z = jnp.dot(x_ref[...], w_ref[...], preferred_element_type=jnp.float32)
    u = dv_ref[...] * z
    for g, o_ref in enumerate(o_refs):
        o_ref[...] = u[:, g * GW:(g + 1) * GW]


def _mm2_body(*refs, G, O, full_parts):
    dv_ref, b_ref, w_ref, o_ref = refs[-4:]
    parts = []
    for g in range(G):
        p_ref, u_ref = refs[2 * g], refs[2 * g + 1]
        if full_parts:
            parts.append(p_ref[...] + u_ref[...])
        else:
            parts.append(p_ref[0] + p_ref[1] + u_ref[...])
    s = jnp.concatenate(parts, axis=1) if G > 1 else parts[0]
    h = jnp.maximum(dv_ref[...] * s + b_ref[...], 0.0)
    z = jnp.dot(h, w_ref[...], preferred_element_type=jnp.float32)
    u = dv_ref[...] * z
    o_ref[...] = u


def _fin_body(q_ref, u_ref, dv_ref, b_ref, o_ref):
    s = q_ref[0] + q_ref[1] + u_ref[...]
    o_ref[...] = dv_ref[...] * s + b_ref[...]


def kernel(x, edge_index, W1, b1, W2, b2):
    N, D = x.shape
    E = edge_index.shape[1]
    H = W1.shape[1]
    O = W2.shape[1]
    G1 = H // GW

    Npad = ((N + 1 + 255) // 256) * 256
    RB = Npad // 8
    K = -(-E // (NW * CHUNK))
    EP = NW * K * CHUNK
    EW = K * CHUNK

    pk_flat = jnp.concatenate([
        edge_index[0].astype(jnp.int32)
        | (edge_index[1].astype(jnp.int32) << 16),
        jnp.full((EP - E,), N << 16, jnp.int32),
    ])

    x_pad = jnp.zeros((Npad, D), jnp.float32).at[:N].set(x)
    zeros_g = jnp.zeros((Npad, GW), jnp.float32)

    deg_part = _make_deg_kernel(Npad, E)(
        edge_index.astype(jnp.int32).reshape(2 * E))

    NR = Npad // 128
    dinv2d = pl.pallas_call(
        functools.partial(_prep_body, N=N),
        out_shape=jax.ShapeDtypeStruct((NR, 128), jnp.float32),
    )(deg_part.reshape(NW, NR, 128))
    dinv = dinv2d.reshape(Npad, 1)

    u1g = pl.pallas_call(
        _mm1_body,
        grid=(Npad // RB,),
        in_specs=[
            pl.BlockSpec((RB, D), lambda i: (i, 0)),
            pl.BlockSpec((D, H), lambda i: (0, 0)),
            pl.BlockSpec((RB, 1), lambda i: (i, 0)),
        ],
        out_specs=[pl.BlockSpec((RB, GW), lambda i: (i, 0))] * G1,
        out_shape=[jax.ShapeDtypeStruct((Npad, GW), jnp.float32)] * G1,
    )(x_pad, W1, dinv)

    full_parts = G1 == 2
    if full_parts:
        K2 = 2 * K
        pg = list(_make_agg2g_kernel(Npad, GW, K2)(
            u1g[0], u1g[1], pk_flat, zeros_g))
    else:
        agg1 = _make_agg_kernel(Npad, GW, K)
        pg = [agg1(u1g[g], pk_flat, zeros_g) for g in range(G1)]

    mm2_ins = []
    mm2_specs = []
    for g in range(G1):
        mm2_ins += [pg[g], u1g[g]]
        if full_parts:
            mm2_specs += [pl.BlockSpec((RB, GW), lambda i: (i, 0))]
        else:
            mm2_specs += [pl.BlockSpec((2, RB, GW), lambda i: (0, i, 0))]
        mm2_specs += [pl.BlockSpec((RB, GW), lambda i: (i, 0))]
    mm2_ins += [dinv, b1.reshape(1, H), W2]
    mm2_specs += [pl.BlockSpec((RB, 1), lambda i: (i, 0)),
                  pl.BlockSpec((1, H), lambda i: (0, 0)),
                  pl.BlockSpec((H, O), lambda i: (0, 0))]
    u2 = pl.pallas_call(
        functools.partial(_mm2_body, G=G1, O=O, full_parts=full_parts),
        grid=(Npad // RB,),
        in_specs=mm2_specs,
        out_specs=pl.BlockSpec((RB, O), lambda i: (i, 0)),
        out_shape=jax.ShapeDtypeStruct((Npad, O), jnp.float32),
    )(*mm2_ins)

    q = _make_agg_kernel(Npad, O, K)(u2, pk_flat, zeros_g)

    RF = N // 10
    out = pl.pallas_call(
        _fin_body,
        grid=(10,),
        in_specs=[
            pl.BlockSpec((2, RF, O), lambda i: (0, i, 0)),
            pl.BlockSpec((RF, O), lambda i: (i, 0)),
            pl.BlockSpec((RF, 1), lambda i: (i, 0)),
            pl.BlockSpec((1, O), lambda i: (0, 0)),
        ],
        out_specs=pl.BlockSpec((RF, O), lambda i: (i, 0)),
        out_shape=jax.ShapeDtypeStruct((N, O), jnp.float32),
    )(q, u2, dinv, b2.reshape(1, O))

    return out
